# baseline (speedup 1.0000x reference)
import functools

import jax
import jax.numpy as jnp
from jax import lax
from jax.experimental import pallas as pl
from jax.experimental.pallas import tpu as pltpu
from jax.experimental.pallas import tpu_sc as plsc

B = 16384
EMB = 64
NC, NS = 2, 16
NW = NC * NS
BPW = B // NW
CH = 128
NCH = BPW // CH
NF = 5


def _sc_gather5(tables, idxs):
    idx3 = [i.reshape(NW, NCH, CH) for i in idxs]
    mesh = plsc.VectorSubcoreMesh(core_axis_name="c", subcore_axis_name="s")
    out_type = tuple(
        jax.ShapeDtypeStruct((NW * NCH, CH, EMB), jnp.float32) for _ in range(NF)
    )
    scratch = (
        [pltpu.VMEM((NCH, CH), jnp.int32) for _ in range(NF)]
        + [pltpu.VMEM((NCH, CH, EMB), jnp.float32) for _ in range(2)]
        + [pltpu.SemaphoreType.DMA for _ in range(2)]
    )

    @functools.partial(
        pl.kernel, out_type=out_type, mesh=mesh, scratch_types=scratch,
        compiler_params=pltpu.CompilerParams(use_tc_tiling_on_sc=False))
    def k(*refs):
        tbls = refs[0:NF]
        idxr = refs[NF:2 * NF]
        outs = refs[2 * NF:3 * NF]
        idx_v = refs[3 * NF:4 * NF]
        bufs = refs[4 * NF:4 * NF + 2]
        sems = refs[4 * NF + 2:4 * NF + 4]

        wid = lax.axis_index("s") * NC + lax.axis_index("c")

        for f in range(NF):
            pltpu.sync_copy(idxr[f].at[wid], idx_v[f])

        def fire(f):
            b = f % 2
            return [
                pltpu.async_copy(tbls[f].at[idx_v[f].at[j]], bufs[b].at[j],
                                 sems[b])
                for j in range(NCH)
            ]

        pending = fire(0)
        for f in range(NF):
            nxt = fire(f + 1) if f + 1 < NF else None
            for h in pending:
                h.wait()
            pltpu.sync_copy(bufs[f % 2], outs[f].at[pl.ds(wid * NCH, NCH)])
            pending = nxt

    outs = k(*tables, *idx3)
    return [o.reshape(B, EMB) for o in outs]


def _tc_towers(g_uid, g_age, g_reg, g_iid, g_cat, text,
               Wt, bt, Wu1a, Wu1b, Wu1c, bu1, Wu2, bu2, Wu3, bu3,
               Wi1a, Wi1b, Wi1c, bi1, Wi2, bi2, Wi3, bi3):
    bm = 1024
    grid = (B // bm,)
    f32 = jnp.float32

    def dot(a, b):
        return lax.dot_general(a, b, (((1,), (0,)), ((), ())),
                               preferred_element_type=f32)

    def body(uid_ref, age_ref, reg_ref, iid_ref, cat_ref, text_ref,
             wt_ref, bt_ref,
             wu1a_ref, wu1b_ref, wu1c_ref, bu1_ref, wu2_ref, bu2_ref,
             wu3_ref, bu3_ref,
             wi1a_ref, wi1b_ref, wi1c_ref, bi1_ref, wi2_ref, bi2_ref,
             wi3_ref, bi3_ref,
             uout_ref, iout_ref):
        h = (dot(uid_ref[...], wu1a_ref[...])
             + dot(age_ref[...], wu1b_ref[...])
             + dot(reg_ref[...], wu1c_ref[...]) + bu1_ref[...])
        h = jnp.maximum(h, 0.0)
        h = jnp.maximum(dot(h, wu2_ref[...]) + bu2_ref[...], 0.0)
        u = dot(h, wu3_ref[...]) + bu3_ref[...]
        n = jnp.sqrt(jnp.sum(u * u, axis=1, keepdims=True))
        uout_ref[...] = u / jnp.maximum(n, 1e-12)
        te = dot(text_ref[...], wt_ref[...]) + bt_ref[...]
        h = (dot(iid_ref[...], wi1a_ref[...])
             + dot(cat_ref[...], wi1b_ref[...])
             + dot(te, wi1c_ref[...]) + bi1_ref[...])
        h = jnp.maximum(h, 0.0)
        h = jnp.maximum(dot(h, wi2_ref[...]) + bi2_ref[...], 0.0)
        v = dot(h, wi3_ref[...]) + bi3_ref[...]
        n = jnp.sqrt(jnp.sum(v * v, axis=1, keepdims=True))
        iout_ref[...] = v / jnp.maximum(n, 1e-12)

    def batch_spec(d):
        return pl.BlockSpec((bm, d), lambda i: (i, 0))

    def full_spec(a):
        return pl.BlockSpec(a.shape, lambda i: (0,) * a.ndim)

    weights = (Wt, bt, Wu1a, Wu1b, Wu1c, bu1, Wu2, bu2, Wu3, bu3,
               Wi1a, Wi1b, Wi1c, bi1, Wi2, bi2, Wi3, bi3)
    in_specs = ([batch_spec(EMB)] * 5 + [batch_spec(text.shape[1])]
                + [full_spec(w) for w in weights])
    out_specs = [batch_spec(EMB), batch_spec(EMB)]
    out_shape = [jax.ShapeDtypeStruct((B, EMB), f32)] * 2

    return pl.pallas_call(
        body, grid=grid, in_specs=in_specs, out_specs=out_specs,
        out_shape=out_shape,
    )(g_uid, g_age, g_reg, g_iid, g_cat, text, *weights)


def kernel(user_id, user_age, user_region, item_id, item_category,
           text_features,
           E_user_id, E_user_age, E_user_region, E_item_id, E_item_category,
           Wt, bt, Wu1, bu1, Wu2, bu2, Wu3, bu3, Wi1, bi1, Wi2, bi2, Wi3, bi3):
    g_uid, g_age, g_reg, g_iid, g_cat = _sc_gather5(
        [E_user_id, E_user_age, E_user_region, E_item_id, E_item_category],
        [user_id, user_age, user_region, item_id, item_category])
    user_out, item_out = _tc_towers(
        g_uid, g_age, g_reg, g_iid, g_cat, text_features,
        Wt, bt.reshape(1, EMB),
        Wu1[0:EMB], Wu1[EMB:2 * EMB], Wu1[2 * EMB:3 * EMB],
        bu1.reshape(1, -1), Wu2, bu2.reshape(1, -1), Wu3, bu3.reshape(1, -1),
        Wi1[0:EMB], Wi1[EMB:2 * EMB], Wi1[2 * EMB:3 * EMB],
        bi1.reshape(1, -1), Wi2, bi2.reshape(1, -1), Wi3, bi3.reshape(1, -1))
    return (user_out, item_out)
